# Initial kernel scaffold; baseline (speedup 1.0000x reference)
import functools

import jax
import jax.numpy as jnp
from jax import lax
from jax.experimental import pallas as pl
from jax.experimental.pallas import tpu as pltpu
from jax.experimental.pallas import tpu_sc as plsc

K = 1024
B2 = 2 * K
NG = B2 // 16
HWORDS = (NG + 1) * 256
NC, NS = 2, 16
NW = NC * NS
PER_W = 12544
CHUNK = 1568
NVEC = CHUNK // 16
NCHUNK = PER_W // CHUNK
NPAD = NW * PER_W

_mesh = plsc.VectorSubcoreMesh(
    core_axis_name="c", subcore_axis_name="s", num_cores=NC, num_subcores=NS
)


@functools.partial(
    pl.kernel,
    out_type=jax.ShapeDtypeStruct((NC, B2), jnp.int32),
    mesh=_mesh,
    scratch_types=[
        pltpu.VMEM((CHUNK,), jnp.int32),
        pltpu.VMEM((HWORDS,), jnp.int32),
        pltpu.VMEM((B2,), jnp.int32),
        pltpu.VMEM((NS, B2), jnp.int32),
        pltpu.VMEM_SHARED((NS, B2), jnp.int32),
    ],
)
def _hist(packed_hbm, t_hbm, buf, hist, tloc, tall, tshared):
    c = lax.axis_index("c")
    s = lax.axis_index("s")
    iot = lax.iota(jnp.int32, 16)
    lane16 = iot * 16
    ones = jnp.ones((16,), jnp.int32)
    zeros = jnp.zeros((16,), jnp.int32)

    def _zero(i, carry):
        hist[pl.ds(i * 16, 16)] = zeros
        return carry

    lax.fori_loop(0, HWORDS // 16, _zero, 0)

    base = (c * NS + s) * PER_W

    def _chunk(ci, carry):
        pltpu.sync_copy(packed_hbm.at[pl.ds(base + ci * CHUNK, CHUNK)], buf)

        def _vec(v, carry2):
            pk = buf[pl.ds(v * 16, 16)]
            lb = pk & 3
            p = plsc.bitcast(pk & ~3, jnp.float32)
            fg = lb == 2
            valid = lb != 0
            e = jnp.where(fg, 1.0 - p, p)
            bi = (e * jnp.float32(K)).astype(jnp.int32)
            bi = jnp.minimum(jnp.maximum(bi, 0), K - 1)
            b2 = bi + jnp.where(fg, K, 0)
            idx = ((b2 >> 4) << 8) + lane16 + (b2 & 15)
            idx = jnp.where(valid, idx, NG * 256 + lane16)
            plsc.addupdate_scatter(hist, [idx], ones)
            return carry2

        lax.fori_loop(0, NVEC, _vec, 0)
        return carry

    lax.fori_loop(0, NCHUNK, _chunk, 0)

    def _fold(g, carry):
        acc = hist[pl.ds(g * 256, 16)]
        for r in range(1, 16):
            acc = acc + hist[pl.ds(g * 256 + r * 16, 16)]
        tloc[pl.ds(g * 16, 16)] = acc
        return carry

    lax.fori_loop(0, NG, _fold, 0)

    pltpu.sync_copy(tloc, tshared.at[s])
    plsc.subcore_barrier()

    @pl.when(s == 0)
    def _():
        pltpu.sync_copy(tshared, tall)

        def _sum(g, carry):
            acc = tall[0, pl.ds(g * 16, 16)]
            for r in range(1, NS):
                acc = acc + tall[r, pl.ds(g * 16, 16)]
            tloc[pl.ds(g * 16, 16)] = acc
            return carry

        lax.fori_loop(0, NG, _sum, 0)
        pltpu.sync_copy(tloc, t_hbm.at[c])


@functools.partial(
    pl.kernel,
    out_type=jax.ShapeDtypeStruct((16,), jnp.float32),
    mesh=_mesh,
    scratch_types=[
        pltpu.VMEM((NC, B2), jnp.int32),
        pltpu.VMEM((16,), jnp.float32),
    ],
)
def _scan(t_hbm, out_hbm, tbuf, obuf):
    c = lax.axis_index("c")
    s = lax.axis_index("s")
    iot = lax.iota(jnp.int32, 16)

    @pl.when((c == 0) & (s == 0))
    def _():
        pltpu.sync_copy(t_hbm, tbuf)

        def _g(g, acc):
            return acc + tbuf[0, pl.ds(K + g * 16, 16)] + tbuf[1, pl.ds(K + g * 16, 16)]

        gacc = lax.fori_loop(0, K // 16, _g, jnp.zeros((16,), jnp.int32))
        G = jnp.sum(gacc)

        def _grp(g, carry):
            F0, F1, jacc, jlast = carry
            gg = K // 16 - 1 - g
            h0 = tbuf[0, pl.ds(gg * 16, 16)] + tbuf[1, pl.ds(gg * 16, 16)]
            h1 = tbuf[0, pl.ds(K + gg * 16, 16)] + tbuf[1, pl.ds(K + gg * 16, 16)]
            h0r = lax.rev(h0, (0,))
            h1r = lax.rev(h1, (0,))
            f0v = F0 + plsc.cumsum(h0r)
            f1v = F1 + plsc.cumsum(h1r)
            den = G + f0v
            j = 1.0 - (G - f1v).astype(jnp.float32) / jnp.maximum(den, 1).astype(
                jnp.float32
            )
            j = jnp.where(den == 0, 0.0, j)
            jlast = jnp.sum(jnp.where(iot == 15, j, 0.0))
            return (F0 + jnp.sum(h0), F1 + jnp.sum(h1), jacc + j, jlast)

        init = (jnp.int32(0), jnp.int32(0), jnp.zeros((16,), jnp.float32),
                jnp.float32(0.0))
        _, _, jacc, jlast = lax.fori_loop(0, K // 16, _grp, init)
        loss = (jnp.sum(jacc) - 0.5 * jlast) * jnp.float32(1.0 / K)
        obuf[...] = jnp.where(iot == 0, loss, 0.0)
        pltpu.sync_copy(obuf, out_hbm)


def kernel(probas, labels):
    p = probas[:, 2, :].reshape(-1)
    lab = labels.reshape(-1).astype(jnp.int32)
    pi = lax.bitcast_convert_type(p, jnp.int32)
    packed = (pi & ~jnp.int32(3)) | lab
    packed = jnp.concatenate(
        [packed, jnp.zeros((NPAD - packed.shape[0],), jnp.int32)]
    )
    t = _hist(packed)
    out16 = _scan(t)
    return out16[0]


# speedup vs baseline: 13.4491x; 13.4491x over previous
import functools

import jax
import jax.numpy as jnp
from jax import lax
from jax.experimental import pallas as pl
from jax.experimental.pallas import tpu as pltpu
from jax.experimental.pallas import tpu_sc as plsc

K = 1024
B2 = 2 * K
NG = B2 // 16
HWORDS = (NG + 1) * 256
NC, NS = 2, 16
NW = NC * NS
PER_W = 12544
CHUNK = 1568
NVEC = CHUNK // 16
NCHUNK = PER_W // CHUNK
NPAD = NW * PER_W

_mesh = plsc.VectorSubcoreMesh(
    core_axis_name="c", subcore_axis_name="s", num_cores=NC, num_subcores=NS
)


@functools.partial(
    pl.kernel,
    out_type=jax.ShapeDtypeStruct((NC, B2), jnp.int32),
    mesh=_mesh,
    scratch_types=[
        pltpu.VMEM((CHUNK,), jnp.int32),
        pltpu.VMEM((HWORDS,), jnp.int32),
        pltpu.VMEM((B2,), jnp.int32),
        pltpu.VMEM((NS, B2), jnp.int32),
        pltpu.VMEM_SHARED((NS, B2), jnp.int32),
    ],
    compiler_params=pltpu.CompilerParams(needs_layout_passes=False),
)
def _hist(packed_hbm, t_hbm, buf, hist, tloc, tall, tshared):
    c = lax.axis_index("c")
    s = lax.axis_index("s")
    iot = lax.iota(jnp.int32, 16)
    lane16 = iot * 16
    ones = jnp.ones((16,), jnp.int32)
    zeros = jnp.zeros((16,), jnp.int32)

    def _zero(i, carry):
        hist[pl.ds(i * 16, 16)] = zeros
        return carry

    lax.fori_loop(0, HWORDS // 16, _zero, 0)

    base = (c * NS + s) * PER_W

    def _chunk(ci, carry):
        pltpu.sync_copy(packed_hbm.at[pl.ds(base + ci * CHUNK, CHUNK)], buf)

        def _vec(v, carry2):
            pk = buf[pl.ds(v * 16, 16)]
            lb = pk & 3
            p = lax.bitcast_convert_type(pk & ~3, jnp.float32)
            fg = lb == 2
            valid = lb != 0
            e = jnp.where(fg, 1.0 - p, p)
            bi = (e * jnp.float32(K)).astype(jnp.int32)
            bi = jnp.minimum(jnp.maximum(bi, 0), K - 1)
            b2 = bi + jnp.where(fg, K, 0)
            idx = ((b2 >> 4) << 8) + lane16 + (b2 & 15)
            idx = jnp.where(valid, idx, NG * 256 + lane16)
            plsc.addupdate_scatter(hist, [idx], ones)
            return carry2

        lax.fori_loop(0, NVEC, _vec, 0)
        return carry

    lax.fori_loop(0, NCHUNK, _chunk, 0)

    def _fold(g, carry):
        acc = hist[pl.ds(g * 256, 16)]
        for r in range(1, 16):
            acc = acc + hist[pl.ds(g * 256 + r * 16, 16)]
        tloc[pl.ds(g * 16, 16)] = acc
        return carry

    lax.fori_loop(0, NG, _fold, 0)

    pltpu.sync_copy(tloc, tshared.at[s])
    plsc.subcore_barrier()

    @pl.when(s == 0)
    def _():
        pltpu.sync_copy(tshared, tall)

        def _sum(g, carry):
            acc = tall[0, pl.ds(g * 16, 16)]
            for r in range(1, NS):
                acc = acc + tall[r, pl.ds(g * 16, 16)]
            tloc[pl.ds(g * 16, 16)] = acc
            return carry

        lax.fori_loop(0, NG, _sum, 0)
        pltpu.sync_copy(tloc, t_hbm.at[c])


@functools.partial(
    pl.kernel,
    out_type=jax.ShapeDtypeStruct((16,), jnp.float32),
    mesh=_mesh,
    scratch_types=[
        pltpu.VMEM((NC, B2), jnp.int32),
        pltpu.VMEM((16,), jnp.float32),
    ],
    compiler_params=pltpu.CompilerParams(needs_layout_passes=False),
)
def _scan(t_hbm, out_hbm, tbuf, obuf):
    c = lax.axis_index("c")
    s = lax.axis_index("s")
    iot = lax.iota(jnp.int32, 16)

    @pl.when((c == 0) & (s == 0))
    def _():
        pltpu.sync_copy(t_hbm, tbuf)

        def _g(g, acc):
            return acc + tbuf[0, pl.ds(K + g * 16, 16)] + tbuf[1, pl.ds(K + g * 16, 16)]

        gacc = lax.fori_loop(0, K // 16, _g, jnp.zeros((16,), jnp.int32))
        G = jnp.sum(gacc)

        def _grp(g, carry):
            F0, F1, jacc, jlast = carry
            gg = K // 16 - 1 - g
            h0 = tbuf[0, pl.ds(gg * 16, 16)] + tbuf[1, pl.ds(gg * 16, 16)]
            h1 = tbuf[0, pl.ds(K + gg * 16, 16)] + tbuf[1, pl.ds(K + gg * 16, 16)]
            h0r = lax.rev(h0, (0,))
            h1r = lax.rev(h1, (0,))
            f0v = F0 + plsc.cumsum(h0r)
            f1v = F1 + plsc.cumsum(h1r)
            den = G + f0v
            j = 1.0 - (G - f1v).astype(jnp.float32) / jnp.maximum(den, 1).astype(
                jnp.float32
            )
            j = jnp.where(den == 0, 0.0, j)
            jlast = jnp.sum(jnp.where(iot == 15, j, 0.0))
            return (F0 + jnp.sum(h0), F1 + jnp.sum(h1), jacc + j, jlast)

        init = (jnp.int32(0), jnp.int32(0), jnp.zeros((16,), jnp.float32),
                jnp.float32(0.0))
        _, _, jacc, jlast = lax.fori_loop(0, K // 16, _grp, init)
        loss = (jnp.sum(jacc) - 0.5 * jlast) * jnp.float32(1.0 / K)
        obuf[...] = jnp.where(iot == 0, loss, 0.0)
        pltpu.sync_copy(obuf, out_hbm)


def kernel(probas, labels):
    p = probas[:, 2, :].reshape(-1)
    lab = labels.reshape(-1).astype(jnp.int32)
    pi = lax.bitcast_convert_type(p, jnp.int32)
    packed = (pi & ~jnp.int32(3)) | lab
    packed = jnp.concatenate(
        [packed, jnp.zeros((NPAD - packed.shape[0],), jnp.int32)]
    )
    t = _hist(packed)
    out16 = _scan(t)
    return out16[0]


# speedup vs baseline: 16.9197x; 1.2580x over previous
import functools

import jax
import jax.numpy as jnp
from jax import lax
from jax.experimental import pallas as pl
from jax.experimental.pallas import tpu as pltpu
from jax.experimental.pallas import tpu_sc as plsc

K = 512
B2 = 2 * K
NG = B2 // 16
HWORDS = (NG + 1) * 256
NC, NS = 2, 16
NW = NC * NS
PER_W = 12544
CHUNK = 1568
NVEC = CHUNK // 16
NCHUNK = PER_W // CHUNK
NPAD = NW * PER_W

_mesh = plsc.VectorSubcoreMesh(
    core_axis_name="c", subcore_axis_name="s", num_cores=NC, num_subcores=NS
)


@functools.partial(
    pl.kernel,
    out_type=jax.ShapeDtypeStruct((NC, B2), jnp.int32),
    mesh=_mesh,
    scratch_types=[
        pltpu.VMEM((CHUNK,), jnp.int32),
        pltpu.VMEM((CHUNK,), jnp.int32),
        pltpu.VMEM((HWORDS,), jnp.int32),
        pltpu.VMEM((B2,), jnp.int32),
        pltpu.VMEM((NS, B2), jnp.int32),
        pltpu.VMEM_SHARED((NS, B2), jnp.int32),
        pltpu.SemaphoreType.DMA,
        pltpu.SemaphoreType.DMA,
    ],
    compiler_params=pltpu.CompilerParams(needs_layout_passes=False),
)
def _hist(packed_hbm, t_hbm, buf0, buf1, hist, tloc, tall, tshared, sem0, sem1):
    c = lax.axis_index("c")
    s = lax.axis_index("s")
    iot = lax.iota(jnp.int32, 16)
    lane16 = iot * 16
    ones = jnp.ones((16,), jnp.int32)
    zeros = jnp.zeros((16,), jnp.int32)

    base = (c * NS + s) * PER_W

    def _copy(ci, buf, sem):
        return pltpu.make_async_copy(
            packed_hbm.at[pl.ds(base + ci * CHUNK, CHUNK)], buf, sem
        )

    _copy(0, buf0, sem0).start()

    def _zero(i, carry):
        hist[pl.ds(i * 16, 16)] = zeros
        return carry

    lax.fori_loop(0, HWORDS // 16, _zero, 0)

    def _point(pk):
        lb = pk & 3
        p = lax.bitcast_convert_type(pk & ~3, jnp.float32)
        fg = lb == 2
        e = jnp.where(fg, 1.0 - p, p)
        bi = (e * jnp.float32(K)).astype(jnp.int32)
        bi = jnp.minimum(bi, K - 1)
        b2 = bi + jnp.where(fg, K, 0)
        idx = ((b2 >> 4) << 8) + lane16 + (b2 & 15)
        idx = jnp.where(lb != 0, idx, NG * 256 + lane16)
        plsc.addupdate_scatter(hist, [idx], ones)

    def _consume(buf):
        def _vec(v, carry2):
            _point(buf[pl.ds(v * 32, 16)])
            _point(buf[pl.ds(v * 32 + 16, 16)])
            return carry2

        lax.fori_loop(0, NVEC // 2, _vec, 0)

    def _pair(i, carry):
        _copy(2 * i + 1, buf1, sem1).start()
        _copy(2 * i, buf0, sem0).wait()
        _consume(buf0)

        @pl.when(2 * i + 2 < NCHUNK)
        def _():
            _copy(2 * i + 2, buf0, sem0).start()

        _copy(2 * i + 1, buf1, sem1).wait()
        _consume(buf1)
        return carry

    lax.fori_loop(0, NCHUNK // 2, _pair, 0)

    def _fold(g, carry):
        acc = hist[pl.ds(g * 256, 16)]
        for r in range(1, 16):
            acc = acc + hist[pl.ds(g * 256 + r * 16, 16)]
        tloc[pl.ds(g * 16, 16)] = acc
        return carry

    lax.fori_loop(0, NG, _fold, 0)

    pltpu.sync_copy(tloc, tshared.at[s])
    plsc.subcore_barrier()

    @pl.when(s == 0)
    def _():
        pltpu.sync_copy(tshared, tall)

        def _sum(g, carry):
            acc = tall[0, pl.ds(g * 16, 16)]
            for r in range(1, NS):
                acc = acc + tall[r, pl.ds(g * 16, 16)]
            tloc[pl.ds(g * 16, 16)] = acc
            return carry

        lax.fori_loop(0, NG, _sum, 0)
        pltpu.sync_copy(tloc, t_hbm.at[c])


@functools.partial(
    pl.kernel,
    out_type=jax.ShapeDtypeStruct((16,), jnp.float32),
    mesh=_mesh,
    scratch_types=[
        pltpu.VMEM((NC, B2), jnp.int32),
        pltpu.VMEM((16,), jnp.float32),
    ],
    compiler_params=pltpu.CompilerParams(needs_layout_passes=False),
)
def _scan(t_hbm, out_hbm, tbuf, obuf):
    c = lax.axis_index("c")
    s = lax.axis_index("s")
    iot = lax.iota(jnp.int32, 16)

    @pl.when((c == 0) & (s == 0))
    def _():
        pltpu.sync_copy(t_hbm, tbuf)

        def _g(g, acc):
            return acc + tbuf[0, pl.ds(K + g * 16, 16)] + tbuf[1, pl.ds(K + g * 16, 16)]

        gacc = lax.fori_loop(0, K // 16, _g, jnp.zeros((16,), jnp.int32))
        G = jnp.sum(gacc)

        def _grp(g, carry):
            F0, F1, jacc, jlast = carry
            gg = K // 16 - 1 - g
            h0 = tbuf[0, pl.ds(gg * 16, 16)] + tbuf[1, pl.ds(gg * 16, 16)]
            h1 = tbuf[0, pl.ds(K + gg * 16, 16)] + tbuf[1, pl.ds(K + gg * 16, 16)]
            h0r = lax.rev(h0, (0,))
            h1r = lax.rev(h1, (0,))
            f0v = F0 + plsc.cumsum(h0r)
            f1v = F1 + plsc.cumsum(h1r)
            den = G + f0v
            j = 1.0 - (G - f1v).astype(jnp.float32) / jnp.maximum(den, 1).astype(
                jnp.float32
            )
            j = jnp.where(den == 0, 0.0, j)
            jlast = jnp.sum(jnp.where(iot == 15, j, 0.0))
            return (F0 + jnp.sum(h0), F1 + jnp.sum(h1), jacc + j, jlast)

        init = (jnp.int32(0), jnp.int32(0), jnp.zeros((16,), jnp.float32),
                jnp.float32(0.0))
        _, _, jacc, jlast = lax.fori_loop(0, K // 16, _grp, init)
        loss = (jnp.sum(jacc) - 0.5 * jlast) * jnp.float32(1.0 / K)
        obuf[...] = jnp.where(iot == 0, loss, 0.0)
        pltpu.sync_copy(obuf, out_hbm)


def kernel(probas, labels):
    p = probas[:, 2, :].reshape(-1)
    lab = labels.reshape(-1).astype(jnp.int32)
    pi = lax.bitcast_convert_type(p, jnp.int32)
    packed = (pi & ~jnp.int32(3)) | lab
    packed = jnp.concatenate(
        [packed, jnp.zeros((NPAD - packed.shape[0],), jnp.int32)]
    )
    t = _hist(packed)
    out16 = _scan(t)
    return out16[0]


# speedup vs baseline: 18.6334x; 1.1013x over previous
import functools

import jax
import jax.numpy as jnp
from jax import lax
from jax.experimental import pallas as pl
from jax.experimental.pallas import tpu as pltpu
from jax.experimental.pallas import tpu_sc as plsc

K = 512
B2 = 2 * K
NG = B2 // 16
HWORDS = (NG + 1) * 256
NC, NS = 2, 16
NW = NC * NS
PER_W = 12544
CHUNK = 1568
NVEC = CHUNK // 16
NCHUNK = PER_W // CHUNK
NPAD = NW * PER_W

_mesh = plsc.VectorSubcoreMesh(
    core_axis_name="c", subcore_axis_name="s", num_cores=NC, num_subcores=NS
)


@functools.partial(
    pl.kernel,
    out_type=jax.ShapeDtypeStruct((NW, B2), jnp.int32),
    mesh=_mesh,
    scratch_types=[
        pltpu.VMEM((CHUNK,), jnp.int32),
        pltpu.VMEM((CHUNK,), jnp.int32),
        pltpu.VMEM((HWORDS,), jnp.int32),
        pltpu.VMEM((B2,), jnp.int32),
        pltpu.SemaphoreType.DMA,
        pltpu.SemaphoreType.DMA,
    ],
    compiler_params=pltpu.CompilerParams(needs_layout_passes=False),
)
def _hist(packed_hbm, t_hbm, buf0, buf1, hist, tloc, sem0, sem1):
    c = lax.axis_index("c")
    s = lax.axis_index("s")
    w = c * NS + s
    iot = lax.iota(jnp.int32, 16)
    lane16 = iot * 16
    ones = jnp.ones((16,), jnp.int32)
    zeros = jnp.zeros((16,), jnp.int32)

    base = w * PER_W

    def _copy(ci, buf, sem):
        return pltpu.make_async_copy(
            packed_hbm.at[pl.ds(base + ci * CHUNK, CHUNK)], buf, sem
        )

    _copy(0, buf0, sem0).start()

    def _zero(i, carry):
        hist[pl.ds(i * 16, 16)] = zeros
        return carry

    lax.fori_loop(0, HWORDS // 16, _zero, 0)

    def _point(pk):
        lb = pk & 3
        p = lax.bitcast_convert_type(pk & ~3, jnp.float32)
        fg = lb == 2
        e = jnp.where(fg, 1.0 - p, p)
        bi = (e * jnp.float32(K)).astype(jnp.int32)
        bi = jnp.minimum(bi, K - 1)
        b2 = bi + jnp.where(fg, K, 0)
        idx = ((b2 >> 4) << 8) + lane16 + (b2 & 15)
        idx = jnp.where(lb != 0, idx, NG * 256 + lane16)
        plsc.addupdate_scatter(hist, [idx], ones)

    def _consume(buf):
        def _vec(v, carry2):
            _point(buf[pl.ds(v * 32, 16)])
            _point(buf[pl.ds(v * 32 + 16, 16)])
            return carry2

        lax.fori_loop(0, NVEC // 2, _vec, 0)

    def _pair(i, carry):
        _copy(2 * i + 1, buf1, sem1).start()
        _copy(2 * i, buf0, sem0).wait()
        _consume(buf0)

        @pl.when(2 * i + 2 < NCHUNK)
        def _():
            _copy(2 * i + 2, buf0, sem0).start()

        _copy(2 * i + 1, buf1, sem1).wait()
        _consume(buf1)
        return carry

    lax.fori_loop(0, NCHUNK // 2, _pair, 0)

    def _fold(g, carry):
        acc = hist[pl.ds(g * 256, 16)]
        for r in range(1, 16):
            acc = acc + hist[pl.ds(g * 256 + r * 16, 16)]
        tloc[pl.ds(g * 16, 16)] = acc
        return carry

    lax.fori_loop(0, NG, _fold, 0)

    pltpu.sync_copy(tloc, t_hbm.at[w])


def _scan_body(t_ref, o_ref):
    h = jnp.sum(t_ref[...].astype(jnp.float32), axis=0, keepdims=True)
    h0 = h[:, :K]
    h1 = h[:, K:]
    bi = lax.broadcasted_iota(jnp.int32, (K, K), 0)
    bj = lax.broadcasted_iota(jnp.int32, (K, K), 1)
    suf = (bi >= bj).astype(jnp.float32)
    f0 = jnp.dot(h0, suf, preferred_element_type=jnp.float32)
    f1 = jnp.dot(h1, suf, preferred_element_type=jnp.float32)
    g = jnp.sum(h1)
    den = g + f0
    j = 1.0 - (g - f1) / jnp.maximum(den, 1.0)
    j = jnp.where(den == 0.0, 0.0, j)
    col = lax.broadcasted_iota(jnp.int32, (1, K), 1)
    jlast = jnp.sum(jnp.where(col == 0, j, 0.0))
    o_ref[0, 0] = (jnp.sum(j) - 0.5 * jlast) * jnp.float32(1.0 / K)


_scan_tc = pl.pallas_call(
    _scan_body,
    out_shape=jax.ShapeDtypeStruct((1, 1), jnp.float32),
    out_specs=pl.BlockSpec(memory_space=pltpu.SMEM),
)


def kernel(probas, labels):
    p = probas[:, 2, :].reshape(-1)
    lab = labels.reshape(-1).astype(jnp.int32)
    pi = lax.bitcast_convert_type(p, jnp.int32)
    packed = (pi & ~jnp.int32(3)) | lab
    packed = jnp.concatenate(
        [packed, jnp.zeros((NPAD - packed.shape[0],), jnp.int32)]
    )
    t = _hist(packed)
    out = _scan_tc(t)
    return out[0, 0]


# speedup vs baseline: 27.3888x; 1.4699x over previous
import functools

import jax
import jax.numpy as jnp
from jax import lax
from jax.experimental import pallas as pl
from jax.experimental.pallas import tpu as pltpu
from jax.experimental.pallas import tpu_sc as plsc

K = 512
B2 = 2 * K
LSTRIDE = B2 + 8
HWORDS = 16 * LSTRIDE
NC, NS = 2, 16
NW = NC * NS
PER_W = 12800
CHUNK = 1600
NVEC = CHUNK // 16
NCHUNK = PER_W // CHUNK
NPAD = NW * PER_W
INV = 0x40004189
SCALE = 511.99

_mesh = plsc.VectorSubcoreMesh(
    core_axis_name="c", subcore_axis_name="s", num_cores=NC, num_subcores=NS
)


@functools.partial(
    pl.kernel,
    out_type=jax.ShapeDtypeStruct((NW, B2), jnp.int32),
    mesh=_mesh,
    scratch_types=[
        pltpu.VMEM((CHUNK,), jnp.int32),
        pltpu.VMEM((CHUNK,), jnp.int32),
        pltpu.VMEM((HWORDS,), jnp.int32),
        pltpu.VMEM((B2,), jnp.int32),
        pltpu.SemaphoreType.DMA,
        pltpu.SemaphoreType.DMA,
    ],
    compiler_params=pltpu.CompilerParams(needs_layout_passes=False),
)
def _hist(packed_hbm, t_hbm, buf0, buf1, hist, tloc, sem0, sem1):
    c = lax.axis_index("c")
    s = lax.axis_index("s")
    w = c * NS + s
    iot = lax.iota(jnp.int32, 16)
    lane_base = iot * LSTRIDE
    ones = jnp.ones((16,), jnp.int32)
    zeros = jnp.zeros((16,), jnp.int32)

    base = w * PER_W

    def _copy(ci, buf, sem):
        return pltpu.make_async_copy(
            packed_hbm.at[pl.ds(base + ci * CHUNK, CHUNK)], buf, sem
        )

    _copy(0, buf0, sem0).start()

    def _zero(i, carry):
        for u in range(8):
            hist[pl.ds(i * 128 + u * 16, 16)] = zeros
        return carry

    lax.fori_loop(0, HWORDS // 128, _zero, 0)

    def _bucket(pk):
        lb = pk & 3
        p = lax.bitcast_convert_type(pk, jnp.float32)
        q = jnp.where(lb == 2, 2.0 - p, p)
        bi = (q * jnp.float32(SCALE)).astype(jnp.int32)
        return bi + lane_base

    def _consume(buf):
        def _vec(v, carry2):
            pks = [buf[pl.ds(v * 64 + 16 * u, 16)] for u in range(4)]
            ixs = [_bucket(pk) for pk in pks]
            for ix in ixs:
                plsc.addupdate_scatter(hist, [ix], ones)
            return carry2

        lax.fori_loop(0, NVEC // 4, _vec, 0)

    def _pair(i, carry):
        _copy(2 * i + 1, buf1, sem1).start()
        _copy(2 * i, buf0, sem0).wait()
        _consume(buf0)

        @pl.when(2 * i + 2 < NCHUNK)
        def _():
            _copy(2 * i + 2, buf0, sem0).start()

        _copy(2 * i + 1, buf1, sem1).wait()
        _consume(buf1)
        return carry

    lax.fori_loop(0, NCHUNK // 2, _pair, 0)

    def _fold(g, carry):
        acc = hist[pl.ds(g * 16, 16)]
        for r in range(1, 16):
            acc = acc + hist[pl.ds(r * LSTRIDE + g * 16, 16)]
        tloc[pl.ds(g * 16, 16)] = acc
        return carry

    lax.fori_loop(0, B2 // 16, _fold, 0)

    pltpu.sync_copy(tloc, t_hbm.at[w])


def _scan_body(t_ref, o_ref):
    h = jnp.sum(t_ref[...].astype(jnp.float32), axis=0, keepdims=True)
    h0 = h[:, :K]
    h1 = h[:, K:]
    bi = lax.broadcasted_iota(jnp.int32, (K, K), 0)
    bj = lax.broadcasted_iota(jnp.int32, (K, K), 1)
    suf = (bi >= bj).astype(jnp.float32)
    f0 = jnp.dot(h0, suf, preferred_element_type=jnp.float32)
    f1 = jnp.dot(h1, suf, preferred_element_type=jnp.float32)
    g = jnp.sum(h1)
    den = g + f0
    j = 1.0 - (g - f1) / jnp.maximum(den, 1.0)
    j = jnp.where(den == 0.0, 0.0, j)
    col = lax.broadcasted_iota(jnp.int32, (1, K), 1)
    jlast = jnp.sum(jnp.where(col == 0, j, 0.0))
    o_ref[0, 0] = (jnp.sum(j) - 0.5 * jlast) * jnp.float32(1.0 / K)


_scan_tc = pl.pallas_call(
    _scan_body,
    out_shape=jax.ShapeDtypeStruct((1, 1), jnp.float32),
    out_specs=pl.BlockSpec(memory_space=pltpu.SMEM),
)


def kernel(probas, labels):
    p = probas[:, 2, :].reshape(-1)
    lab = labels.reshape(-1).astype(jnp.int32)
    pi = lax.bitcast_convert_type(p, jnp.int32)
    packed = jnp.where(lab == 0, jnp.int32(INV), (pi & ~jnp.int32(3)) | lab)
    packed = jnp.concatenate(
        [packed, jnp.full((NPAD - packed.shape[0],), INV, jnp.int32)]
    )
    t = _hist(packed)
    out = _scan_tc(t)
    return out[0, 0]
